# baseline (speedup 1.0000x reference)
import functools

import jax
import jax.numpy as jnp
from jax import lax
from jax.experimental import pallas as pl
from jax.experimental.pallas import tpu as pltpu
from jax.experimental.pallas import tpu_sc as plsc

_f32 = jnp.float32

NC, NS = 2, 16
NW = NC * NS

NP = 10240
WR = NP // 128
NG = 8
NACC = NP + NG * WR


def _tables_body(nodes_ref, wall_ref, ts_ref, tr_ref, qr_ref):
    t = jnp.dot(nodes_ref[...], wall_ref[...], preferred_element_type=_f32)
    ts_ref[...] = t[:, :256]
    tr_ref[...] = t[:, 256:384]
    qr_ref[...] = t[:, 384:512]


def _node_tables(nodes, wall, n_blk):
    n, d = nodes.shape
    return pl.pallas_call(
        _tables_body,
        grid=(n // n_blk,),
        in_specs=[
            pl.BlockSpec((n_blk, d), lambda i: (i, 0)),
            pl.BlockSpec((d, 512), lambda i: (0, 0)),
        ],
        out_specs=[
            pl.BlockSpec((n_blk, 256), lambda i: (i, 0)),
            pl.BlockSpec((n_blk, 128), lambda i: (i, 0)),
            pl.BlockSpec((n_blk, 128), lambda i: (i, 0)),
        ],
        out_shape=[
            jax.ShapeDtypeStruct((n, 256), _f32),
            jax.ShapeDtypeStruct((n, 128), _f32),
            jax.ShapeDtypeStruct((n, 128), _f32),
        ],
    )(nodes, wall)


def _sc_gather(ts, tr, snd, rcv, chunk):
    e = snd.shape[0]
    per_w = e // NW
    chunks = per_w // chunk
    mesh = plsc.VectorSubcoreMesh(
        core_axis_name="c", subcore_axis_name="s", num_cores=NC, num_subcores=NS)

    @functools.partial(
        pl.kernel,
        out_type=[jax.ShapeDtypeStruct((e, 256), _f32),
                  jax.ShapeDtypeStruct((e, 128), _f32)],
        mesh=mesh,
        scratch_types=[
            pltpu.VMEM((chunk,), jnp.int32),
            pltpu.VMEM((chunk,), jnp.int32),
            pltpu.VMEM((chunk,), jnp.int32),
            pltpu.VMEM((chunk,), jnp.int32),
            pltpu.VMEM((chunk, 256), _f32),
            pltpu.VMEM((chunk, 256), _f32),
            pltpu.VMEM((chunk, 128), _f32),
            pltpu.VMEM((chunk, 128), _f32),
            pltpu.SemaphoreType.DMA,
            pltpu.SemaphoreType.DMA,
        ],
    )
    def k(ts_hbm, tr_hbm, snd_hbm, rcv_hbm, gs_hbm, gr_hbm,
          is0, is1, ir0, ir1, bs0, bs1, br0, br1, sem_s, sem_r):
        wid = lax.axis_index("s") * NC + lax.axis_index("c")
        w_base = wid * per_w
        idx_s = (is0, is1)
        idx_r = (ir0, ir1)
        bufs_s = (bs0, bs1)
        bufs_r = (br0, br1)

        def start(i, b):
            pltpu.sync_copy(snd_hbm.at[pl.ds(w_base + i * chunk, chunk)], idx_s[b])
            pltpu.sync_copy(rcv_hbm.at[pl.ds(w_base + i * chunk, chunk)], idx_r[b])
            pltpu.async_copy(ts_hbm.at[idx_s[b]], bufs_s[b], sem_s)
            pltpu.async_copy(tr_hbm.at[idx_r[b]], bufs_r[b], sem_r)

        def wait(b):
            pltpu.make_async_copy(ts_hbm.at[idx_s[b]], bufs_s[b], sem_s).wait()
            pltpu.make_async_copy(tr_hbm.at[idx_r[b]], bufs_r[b], sem_r).wait()

        def writeback(i, b):
            pltpu.sync_copy(bufs_s[b], gs_hbm.at[pl.ds(w_base + i * chunk, chunk)])
            pltpu.sync_copy(bufs_r[b], gr_hbm.at[pl.ds(w_base + i * chunk, chunk)])

        start(0, 0)

        @pl.loop(0, chunks, step=2)
        def _main(i0):
            for b in range(2):
                i = i0 + b

                wait(b)

                @pl.when(i + 1 < chunks)
                def _next():
                    start(i + 1, 1 - b)

                writeback(i, b)

    return k(ts, tr, snd, rcv)


def _edge_body(gs_ref, gr_ref, ed_ref, rcv_ref, wqe_ref, w1e_ref, b1_ref,
               gam_ref, bet_ref, w2_ref, b2_ref, outa_ref, outb_ref):
    gs = gs_ref[...]
    eq = jnp.dot(ed_ref[...], wqe_ref[...], preferred_element_type=_f32)
    eh = jnp.dot(ed_ref[...], w1e_ref[...], preferred_element_type=_f32)
    q = gs[:, :128] + eq
    hpre = gs[:, 128:] + gr_ref[...] + eh + b1_ref[...]
    h = jnp.maximum(hpre, 0.0)
    mu = jnp.mean(h, axis=1, keepdims=True)
    var = jnp.mean((h - mu) * (h - mu), axis=1, keepdims=True)
    hln = (h - mu) / jnp.sqrt(var + 1e-5) * gam_ref[...] + bet_ref[...]
    lg = jnp.sum(hln * w2_ref[...], axis=1, keepdims=True) + b2_ref[...]
    w = jnp.exp(jnp.maximum(lg, 0.0))
    lane = lax.broadcasted_iota(jnp.int32, (1, 128), 1)
    oh = (lane == (rcv_ref[...] % 128)).astype(_f32)
    outa_ref[...] = q * w
    outb_ref[...] = w * oh


def _edge_compute(gs, gr, edges, rcv2d, wqe_t, w1e_t, b1, gamma, beta, w2, b2,
                  e_blk):
    e, de = edges.shape
    smalls = [wqe_t, w1e_t, b1.reshape(1, 128), gamma.reshape(1, 128),
              beta.reshape(1, 128), w2.reshape(1, 128), b2.reshape(1, 1)]
    small_specs = [
        pl.BlockSpec((de, 128), lambda i: (0, 0)),
        pl.BlockSpec((de, 128), lambda i: (0, 0)),
        pl.BlockSpec((1, 128), lambda i: (0, 0)),
        pl.BlockSpec((1, 128), lambda i: (0, 0)),
        pl.BlockSpec((1, 128), lambda i: (0, 0)),
        pl.BlockSpec((1, 128), lambda i: (0, 0)),
        pl.BlockSpec((1, 1), lambda i: (0, 0)),
    ]
    return pl.pallas_call(
        _edge_body,
        grid=(e // e_blk,),
        in_specs=[
            pl.BlockSpec((e_blk, 256), lambda i: (i, 0)),
            pl.BlockSpec((e_blk, 128), lambda i: (i, 0)),
            pl.BlockSpec((e_blk, de), lambda i: (i, 0)),
            pl.BlockSpec((e_blk, 1), lambda i: (i, 0)),
        ] + small_specs,
        out_specs=[pl.BlockSpec((e_blk, 128), lambda i: (i, 0)),
                   pl.BlockSpec((e_blk, 128), lambda i: (i, 0))],
        out_shape=[jax.ShapeDtypeStruct((e, 128), _f32),
                   jax.ShapeDtypeStruct((e, 128), _f32)],
    )(gs, gr, edges, rcv2d, *smalls)


def _sc_scatter(msga, msgb, rcv, chunk):
    e = rcv.shape[0]
    per_w = e // NW
    chunks = per_w // chunk
    zr = 40
    per_tile_rows = NACC // NS
    zcopies = per_tile_rows // zr
    mesh = plsc.VectorSubcoreMesh(
        core_axis_name="c", subcore_axis_name="s", num_cores=NC, num_subcores=NS)

    @functools.partial(
        pl.kernel,
        out_type=jax.ShapeDtypeStruct((NC, NACC, 128), _f32),
        mesh=mesh,
        scratch_types=[
            pltpu.VMEM((chunk,), jnp.int32),
            pltpu.VMEM((chunk,), jnp.int32),
            pltpu.VMEM((chunk,), jnp.int32),
            pltpu.VMEM((chunk,), jnp.int32),
            pltpu.VMEM((chunk, 128), _f32),
            pltpu.VMEM((chunk, 128), _f32),
            pltpu.VMEM((chunk, 128), _f32),
            pltpu.VMEM((chunk, 128), _f32),
            pltpu.VMEM((zr, 128), _f32),
            pltpu.VMEM_SHARED((NACC, 128), _f32),
            pltpu.SemaphoreType.DMA,
            pltpu.SemaphoreType.DMA,
        ],
    )
    def k(msga_hbm, msgb_hbm, rcv_hbm, out_hbm,
          ia0, ia1, ib0, ib1, ba0, ba1, bb0, bb1, zbuf, acc, sem_a, sem_b):
        cid = lax.axis_index("c")
        sid = lax.axis_index("s")
        wid = sid * NC + cid
        w_base = wid * per_w
        wrow0 = NP + (sid // 2) * WR
        zeros16 = jnp.zeros((16,), _f32)
        idxa = (ia0, ia1)
        idxb = (ib0, ib1)
        bufa = (ba0, ba1)
        bufb = (bb0, bb1)

        @pl.loop(0, zr)
        def _zrow(r):
            for kk in range(128 // 16):
                zbuf[r, pl.ds(kk * 16, 16)] = zeros16

        @pl.loop(0, zcopies)
        def _zcp(zi):
            pltpu.sync_copy(zbuf, acc.at[pl.ds(sid * per_tile_rows + zi * zr, zr)])

        plsc.subcore_barrier()

        offs = sorted({min(kk * 16, chunk - 16) for kk in range((chunk + 15) // 16)})

        def load_idx(i, b):
            pltpu.sync_copy(rcv_hbm.at[pl.ds(w_base + i * chunk, chunk)], idxa[b])
            for off in offs:
                v = idxa[b][pl.ds(off, 16)]
                idxb[b][pl.ds(off, 16)] = wrow0 + lax.shift_right_logical(v, 7)

        def start_msg(i, b):
            pltpu.async_copy(
                msga_hbm.at[pl.ds(w_base + i * chunk, chunk)], bufa[b], sem_a)
            pltpu.async_copy(
                msgb_hbm.at[pl.ds(w_base + i * chunk, chunk)], bufb[b], sem_b)

        def wait_msg(b):
            pltpu.make_async_copy(
                msga_hbm.at[pl.ds(0, chunk)], bufa[b], sem_a).wait()
            pltpu.make_async_copy(
                msgb_hbm.at[pl.ds(0, chunk)], bufb[b], sem_b).wait()

        def scatter(b):
            pltpu.sync_copy(bufa[b], acc.at[idxa[b]], add=True)
            pltpu.sync_copy(bufb[b], acc.at[idxb[b]], add=True)

        load_idx(0, 0)
        start_msg(0, 0)

        @pl.loop(0, chunks, step=2)
        def _main(i0):
            for b in range(2):
                i = i0 + b

                wait_msg(b)

                @pl.when(i + 1 < chunks)
                def _next():
                    load_idx(i + 1, 1 - b)
                    start_msg(i + 1, 1 - b)

                scatter(b)

        plsc.subcore_barrier()

        @pl.loop(0, zcopies)
        def _dump(zi):
            row = sid * per_tile_rows + zi * zr
            pltpu.sync_copy(acc.at[pl.ds(row, zr)], out_hbm.at[cid, pl.ds(row, zr)])

    return k(msga, msgb, rcv)


def _final_body(parts_ref, wtail_ref, qr_ref, bq_ref, out_ref):
    num = jnp.sum(parts_ref[...], axis=0)
    ws = jnp.sum(wtail_ref[...], axis=0)
    val = (num + ws * (qr_ref[...] + bq_ref[...])) / (ws + 1e-10)
    out_ref[...] = jnp.where(val >= 0.0, val, 0.01 * val)


def _finalize(parts, wtail, qr, bq, n_blk):
    n = qr.shape[0]
    np_parts = parts.shape[0]
    nw_parts = wtail.shape[0]
    return pl.pallas_call(
        _final_body,
        grid=(n // n_blk,),
        in_specs=[
            pl.BlockSpec((np_parts, n_blk, 128), lambda i: (0, i, 0)),
            pl.BlockSpec((nw_parts, n_blk, 1), lambda i: (0, i, 0)),
            pl.BlockSpec((n_blk, 128), lambda i: (i, 0)),
            pl.BlockSpec((1, 128), lambda i: (0, 0)),
        ],
        out_specs=pl.BlockSpec((n_blk, 128), lambda i: (i, 0)),
        out_shape=jax.ShapeDtypeStruct((n, 128), _f32),
    )(parts, wtail, qr, bq.reshape(1, 128))


def kernel(nodes, edge_index, edges, Wq, bq, W1, b1, gamma, beta, W2, b2):
    n, d = nodes.shape
    e = edge_index.shape[1]
    d_e = edges.shape[1]
    wall = jnp.concatenate(
        [Wq[:, :d].T, W1[:, :d].T, W1[:, d:2 * d].T, Wq[:, d:2 * d].T], axis=1)
    wqe_t = Wq[:, 2 * d:].T
    w1e_t = W1[:, 2 * d:].T
    snd = edge_index[0]
    rcv = edge_index[1]
    nodes_p = jnp.pad(nodes, ((0, NP - n), (0, 0)))

    ts, tr, qr = _node_tables(nodes_p, wall, 2048)
    nchunk = 5
    ec = e // nchunk
    parts_l = []
    for c in range(nchunk):
        snd_c = lax.slice(snd, (c * ec,), ((c + 1) * ec,))
        rcv_c = lax.slice(rcv, (c * ec,), ((c + 1) * ec,))
        edges_c = lax.slice(edges, (c * ec, 0), ((c + 1) * ec, d_e))
        gs, gr = _sc_gather(ts, tr, snd_c, rcv_c, 40)
        msga, msgb = _edge_compute(gs, gr, edges_c, rcv_c.reshape(ec, 1),
                                   wqe_t, w1e_t, b1, gamma, beta, W2[0], b2,
                                   2000)
        parts_l.append(_sc_scatter(msga, msgb, rcv_c, 40))
    parts = jnp.concatenate(parts_l, axis=0)
    wtail = parts[:, NP:].reshape(nchunk * NC * NG, NP, 1)
    out = _finalize(parts, wtail, qr, bq, 256)
    return out[:n]


# speedup vs baseline: 1.2107x; 1.2107x over previous
import functools

import jax
import jax.numpy as jnp
from jax import lax
from jax.experimental import pallas as pl
from jax.experimental.pallas import tpu as pltpu
from jax.experimental.pallas import tpu_sc as plsc

_f32 = jnp.float32

NC, NS = 2, 16
NW = NC * NS

NP = 10240
WR = NP // 128
NG = 8
NACC = NP + NG * WR


def _tables_body(nodes_ref, wall_ref, ts_ref, tr_ref, qr_ref):
    t = jnp.dot(nodes_ref[...], wall_ref[...], preferred_element_type=_f32)
    ts_ref[...] = t[:, :256]
    tr_ref[...] = t[:, 256:384]
    qr_ref[...] = t[:, 384:512]


def _node_tables(nodes, wall, n_blk):
    n, d = nodes.shape
    return pl.pallas_call(
        _tables_body,
        grid=(n // n_blk,),
        in_specs=[
            pl.BlockSpec((n_blk, d), lambda i: (i, 0)),
            pl.BlockSpec((d, 512), lambda i: (0, 0)),
        ],
        out_specs=[
            pl.BlockSpec((n_blk, 256), lambda i: (i, 0)),
            pl.BlockSpec((n_blk, 128), lambda i: (i, 0)),
            pl.BlockSpec((n_blk, 128), lambda i: (i, 0)),
        ],
        out_shape=[
            jax.ShapeDtypeStruct((n, 256), _f32),
            jax.ShapeDtypeStruct((n, 128), _f32),
            jax.ShapeDtypeStruct((n, 128), _f32),
        ],
    )(nodes, wall)


def _sc_gather(ts, tr, snd, rcv, chunk):
    e = snd.shape[0]
    per_w = e // NW
    chunks = per_w // chunk
    mesh = plsc.VectorSubcoreMesh(
        core_axis_name="c", subcore_axis_name="s", num_cores=NC, num_subcores=NS)

    @functools.partial(
        pl.kernel,
        out_type=[jax.ShapeDtypeStruct((e, 256), _f32),
                  jax.ShapeDtypeStruct((e, 128), _f32)],
        mesh=mesh,
        scratch_types=[
            pltpu.VMEM((chunk,), jnp.int32),
            pltpu.VMEM((chunk,), jnp.int32),
            pltpu.VMEM((chunk,), jnp.int32),
            pltpu.VMEM((chunk,), jnp.int32),
            pltpu.VMEM((chunk, 256), _f32),
            pltpu.VMEM((chunk, 256), _f32),
            pltpu.VMEM((chunk, 128), _f32),
            pltpu.VMEM((chunk, 128), _f32),
            pltpu.SemaphoreType.DMA,
            pltpu.SemaphoreType.DMA,
        ],
    )
    def k(ts_hbm, tr_hbm, snd_hbm, rcv_hbm, gs_hbm, gr_hbm,
          is0, is1, ir0, ir1, bs0, bs1, br0, br1, sem_s, sem_r):
        wid = lax.axis_index("s") * NC + lax.axis_index("c")
        w_base = wid * per_w
        idx_s = (is0, is1)
        idx_r = (ir0, ir1)
        bufs_s = (bs0, bs1)
        bufs_r = (br0, br1)

        def start(i, b):
            pltpu.sync_copy(snd_hbm.at[pl.ds(w_base + i * chunk, chunk)], idx_s[b])
            pltpu.sync_copy(rcv_hbm.at[pl.ds(w_base + i * chunk, chunk)], idx_r[b])
            pltpu.async_copy(ts_hbm.at[idx_s[b]], bufs_s[b], sem_s)
            pltpu.async_copy(tr_hbm.at[idx_r[b]], bufs_r[b], sem_r)

        def wait(b):
            pltpu.make_async_copy(ts_hbm.at[idx_s[b]], bufs_s[b], sem_s).wait()
            pltpu.make_async_copy(tr_hbm.at[idx_r[b]], bufs_r[b], sem_r).wait()

        def writeback(i, b):
            pltpu.sync_copy(bufs_s[b], gs_hbm.at[pl.ds(w_base + i * chunk, chunk)])
            pltpu.sync_copy(bufs_r[b], gr_hbm.at[pl.ds(w_base + i * chunk, chunk)])

        start(0, 0)

        @pl.loop(0, chunks, step=2)
        def _main(i0):
            for b in range(2):
                i = i0 + b

                wait(b)

                @pl.when(i + 1 < chunks)
                def _next():
                    start(i + 1, 1 - b)

                writeback(i, b)

    return k(ts, tr, snd, rcv)


def _edge_body(gs_ref, gr_ref, ed_ref, rcv_ref, wqe_ref, w1e_ref, b1_ref,
               gam_ref, bet_ref, w2_ref, b2_ref, outa_ref, outb_ref):
    gs = gs_ref[...]
    eq = jnp.dot(ed_ref[...], wqe_ref[...], preferred_element_type=_f32)
    eh = jnp.dot(ed_ref[...], w1e_ref[...], preferred_element_type=_f32)
    q = gs[:, :128] + eq
    hpre = gs[:, 128:] + gr_ref[...] + eh + b1_ref[...]
    h = jnp.maximum(hpre, 0.0)
    mu = jnp.mean(h, axis=1, keepdims=True)
    var = jnp.mean((h - mu) * (h - mu), axis=1, keepdims=True)
    hln = (h - mu) / jnp.sqrt(var + 1e-5) * gam_ref[...] + bet_ref[...]
    lg = jnp.sum(hln * w2_ref[...], axis=1, keepdims=True) + b2_ref[...]
    w = jnp.exp(jnp.maximum(lg, 0.0))
    lane = lax.broadcasted_iota(jnp.int32, (1, 128), 1)
    oh = (lane == (rcv_ref[...] % 128)).astype(_f32)
    outa_ref[...] = q * w
    outb_ref[...] = w * oh


def _edge_compute(gs, gr, edges, rcv2d, wqe_t, w1e_t, b1, gamma, beta, w2, b2,
                  e_blk):
    e, de = edges.shape
    smalls = [wqe_t, w1e_t, b1.reshape(1, 128), gamma.reshape(1, 128),
              beta.reshape(1, 128), w2.reshape(1, 128), b2.reshape(1, 1)]
    small_specs = [
        pl.BlockSpec((de, 128), lambda i: (0, 0)),
        pl.BlockSpec((de, 128), lambda i: (0, 0)),
        pl.BlockSpec((1, 128), lambda i: (0, 0)),
        pl.BlockSpec((1, 128), lambda i: (0, 0)),
        pl.BlockSpec((1, 128), lambda i: (0, 0)),
        pl.BlockSpec((1, 128), lambda i: (0, 0)),
        pl.BlockSpec((1, 1), lambda i: (0, 0)),
    ]
    return pl.pallas_call(
        _edge_body,
        grid=(e // e_blk,),
        in_specs=[
            pl.BlockSpec((e_blk, 256), lambda i: (i, 0)),
            pl.BlockSpec((e_blk, 128), lambda i: (i, 0)),
            pl.BlockSpec((e_blk, de), lambda i: (i, 0)),
            pl.BlockSpec((e_blk, 1), lambda i: (i, 0)),
        ] + small_specs,
        out_specs=[pl.BlockSpec((e_blk, 128), lambda i: (i, 0)),
                   pl.BlockSpec((e_blk, 128), lambda i: (i, 0))],
        out_shape=[jax.ShapeDtypeStruct((e, 128), _f32),
                   jax.ShapeDtypeStruct((e, 128), _f32)],
    )(gs, gr, edges, rcv2d, *smalls)


def _sc_scatter(msga, msgb, rcv, chunk):
    e = rcv.shape[0]
    per_w = e // NW
    chunks = per_w // chunk
    zr = 40
    per_tile_rows = NACC // NS
    zcopies = per_tile_rows // zr
    mesh = plsc.VectorSubcoreMesh(
        core_axis_name="c", subcore_axis_name="s", num_cores=NC, num_subcores=NS)

    @functools.partial(
        pl.kernel,
        out_type=jax.ShapeDtypeStruct((NC, NACC, 128), _f32),
        mesh=mesh,
        scratch_types=[
            pltpu.VMEM((chunk,), jnp.int32),
            pltpu.VMEM((chunk,), jnp.int32),
            pltpu.VMEM((chunk,), jnp.int32),
            pltpu.VMEM((chunk,), jnp.int32),
            pltpu.VMEM((chunk, 128), _f32),
            pltpu.VMEM((chunk, 128), _f32),
            pltpu.VMEM((chunk, 128), _f32),
            pltpu.VMEM((chunk, 128), _f32),
            pltpu.VMEM((zr, 128), _f32),
            pltpu.VMEM_SHARED((NACC, 128), _f32),
            pltpu.SemaphoreType.DMA,
            pltpu.SemaphoreType.DMA,
        ],
    )
    def k(msga_hbm, msgb_hbm, rcv_hbm, out_hbm,
          ia0, ia1, ib0, ib1, ba0, ba1, bb0, bb1, zbuf, acc, sem_a, sem_b):
        cid = lax.axis_index("c")
        sid = lax.axis_index("s")
        wid = sid * NC + cid
        w_base = wid * per_w
        wrow0 = NP + (sid // 2) * WR
        zeros16 = jnp.zeros((16,), _f32)
        idxa = (ia0, ia1)
        idxb = (ib0, ib1)
        bufa = (ba0, ba1)
        bufb = (bb0, bb1)

        @pl.loop(0, zr)
        def _zrow(r):
            for kk in range(128 // 16):
                zbuf[r, pl.ds(kk * 16, 16)] = zeros16

        @pl.loop(0, zcopies)
        def _zcp(zi):
            pltpu.sync_copy(zbuf, acc.at[pl.ds(sid * per_tile_rows + zi * zr, zr)])

        plsc.subcore_barrier()

        offs = sorted({min(kk * 16, chunk - 16) for kk in range((chunk + 15) // 16)})

        def load_idx(i, b):
            pltpu.sync_copy(rcv_hbm.at[pl.ds(w_base + i * chunk, chunk)], idxa[b])
            for off in offs:
                v = idxa[b][pl.ds(off, 16)]
                idxb[b][pl.ds(off, 16)] = wrow0 + lax.shift_right_logical(v, 7)

        def start_msg(i, b):
            pltpu.async_copy(
                msga_hbm.at[pl.ds(w_base + i * chunk, chunk)], bufa[b], sem_a)
            pltpu.async_copy(
                msgb_hbm.at[pl.ds(w_base + i * chunk, chunk)], bufb[b], sem_b)

        def wait_msg(b):
            pltpu.make_async_copy(
                msga_hbm.at[pl.ds(0, chunk)], bufa[b], sem_a).wait()
            pltpu.make_async_copy(
                msgb_hbm.at[pl.ds(0, chunk)], bufb[b], sem_b).wait()

        def scatter(b):
            pltpu.sync_copy(bufa[b], acc.at[idxa[b]], add=True)
            pltpu.sync_copy(bufb[b], acc.at[idxb[b]], add=True)

        load_idx(0, 0)
        start_msg(0, 0)

        @pl.loop(0, chunks, step=2)
        def _main(i0):
            for b in range(2):
                i = i0 + b

                wait_msg(b)

                @pl.when(i + 1 < chunks)
                def _next():
                    load_idx(i + 1, 1 - b)
                    start_msg(i + 1, 1 - b)

                scatter(b)

        plsc.subcore_barrier()

        @pl.loop(0, zcopies)
        def _dump(zi):
            row = sid * per_tile_rows + zi * zr
            pltpu.sync_copy(acc.at[pl.ds(row, zr)], out_hbm.at[cid, pl.ds(row, zr)])

    return k(msga, msgb, rcv)


def _final_body(parts_ref, wtail_ref, qr_ref, bq_ref, out_ref):
    num = jnp.sum(parts_ref[...], axis=0)
    ws = jnp.sum(wtail_ref[...], axis=0)
    val = (num + ws * (qr_ref[...] + bq_ref[...])) / (ws + 1e-10)
    out_ref[...] = jnp.where(val >= 0.0, val, 0.01 * val)


def _finalize(parts, wtail, qr, bq, n_blk):
    n = qr.shape[0]
    np_parts = parts.shape[0]
    nw_parts = wtail.shape[0]
    return pl.pallas_call(
        _final_body,
        grid=(n // n_blk,),
        in_specs=[
            pl.BlockSpec((np_parts, n_blk, 128), lambda i: (0, i, 0)),
            pl.BlockSpec((nw_parts, n_blk, 1), lambda i: (0, i, 0)),
            pl.BlockSpec((n_blk, 128), lambda i: (i, 0)),
            pl.BlockSpec((1, 128), lambda i: (0, 0)),
        ],
        out_specs=pl.BlockSpec((n_blk, 128), lambda i: (i, 0)),
        out_shape=jax.ShapeDtypeStruct((n, 128), _f32),
    )(parts, wtail, qr, bq.reshape(1, 128))


def kernel(nodes, edge_index, edges, Wq, bq, W1, b1, gamma, beta, W2, b2):
    n, d = nodes.shape
    e = edge_index.shape[1]
    d_e = edges.shape[1]
    wall = jnp.concatenate(
        [Wq[:, :d].T, W1[:, :d].T, W1[:, d:2 * d].T, Wq[:, d:2 * d].T], axis=1)
    wqe_t = Wq[:, 2 * d:].T
    w1e_t = W1[:, 2 * d:].T
    snd = edge_index[0]
    rcv = edge_index[1]
    nodes_p = jnp.pad(nodes, ((0, NP - n), (0, 0)))

    ts, tr, qr = _node_tables(nodes_p, wall, 2048)
    nchunk = 1
    ec = e // nchunk
    parts_l = []
    for c in range(nchunk):
        snd_c = lax.slice(snd, (c * ec,), ((c + 1) * ec,))
        rcv_c = lax.slice(rcv, (c * ec,), ((c + 1) * ec,))
        edges_c = lax.slice(edges, (c * ec, 0), ((c + 1) * ec, d_e))
        gs, gr = _sc_gather(ts, tr, snd_c, rcv_c, 40)
        msga, msgb = _edge_compute(gs, gr, edges_c, rcv_c.reshape(ec, 1),
                                   wqe_t, w1e_t, b1, gamma, beta, W2[0], b2,
                                   2000)
        parts_l.append(_sc_scatter(msga, msgb, rcv_c, 40))
    parts = jnp.concatenate(parts_l, axis=0)
    wtail = parts[:, NP:].reshape(nchunk * NC * NG, NP, 1)
    out = _finalize(parts, wtail, qr, bq, 256)
    return out[:n]


# speedup vs baseline: 1.4510x; 1.1985x over previous
import functools

import jax
import jax.numpy as jnp
from jax import lax
from jax.experimental import pallas as pl
from jax.experimental.pallas import tpu as pltpu
from jax.experimental.pallas import tpu_sc as plsc

_f32 = jnp.float32

NC, NS = 2, 16
NW = NC * NS

NP = 10240
WR = NP // 128


def _pack2(hi, lo):
    hb = lax.bitcast_convert_type(hi.astype(jnp.bfloat16), jnp.uint16)
    lb = lax.bitcast_convert_type(lo.astype(jnp.bfloat16), jnp.uint16)
    word = (hb.astype(jnp.uint32) << 16) | lb.astype(jnp.uint32)
    return lax.bitcast_convert_type(word, jnp.int32)


def _unpack_hi(w):
    return lax.bitcast_convert_type(
        lax.bitcast_convert_type(w, jnp.uint32) & jnp.uint32(0xFFFF0000), _f32)


def _unpack_lo(w):
    return lax.bitcast_convert_type(
        lax.bitcast_convert_type(w, jnp.uint32) << 16, _f32)


def _tables_body(nodes_ref, wall_ref, ts_ref, tr_ref, qr_ref):
    t = jnp.dot(nodes_ref[...], wall_ref[...], preferred_element_type=_f32)
    ts_ref[...] = _pack2(t[:, :128], t[:, 128:256])
    tr_ref[...] = t[:, 256:384]
    qr_ref[...] = t[:, 384:512]


def _node_tables(nodes, wall, n_blk):
    n, d = nodes.shape
    return pl.pallas_call(
        _tables_body,
        grid=(n // n_blk,),
        in_specs=[
            pl.BlockSpec((n_blk, d), lambda i: (i, 0)),
            pl.BlockSpec((d, 512), lambda i: (0, 0)),
        ],
        out_specs=[
            pl.BlockSpec((n_blk, 128), lambda i: (i, 0)),
            pl.BlockSpec((n_blk, 128), lambda i: (i, 0)),
            pl.BlockSpec((n_blk, 128), lambda i: (i, 0)),
        ],
        out_shape=[
            jax.ShapeDtypeStruct((n, 128), jnp.int32),
            jax.ShapeDtypeStruct((n, 128), _f32),
            jax.ShapeDtypeStruct((n, 128), _f32),
        ],
    )(nodes, wall)


def _sc_gather(ts, tr, snd, rcv, chunk):
    e = snd.shape[0]
    per_w = e // NW
    chunks = per_w // chunk
    mesh = plsc.VectorSubcoreMesh(
        core_axis_name="c", subcore_axis_name="s", num_cores=NC, num_subcores=NS)

    @functools.partial(
        pl.kernel,
        out_type=[jax.ShapeDtypeStruct((e, 128), jnp.int32),
                  jax.ShapeDtypeStruct((e, 128), _f32)],
        mesh=mesh,
        scratch_types=[
            pltpu.VMEM((chunk,), jnp.int32),
            pltpu.VMEM((chunk,), jnp.int32),
            pltpu.VMEM((chunk, 128), jnp.int32),
            pltpu.VMEM((chunk, 128), _f32),
            pltpu.SemaphoreType.DMA,
            pltpu.SemaphoreType.DMA,
        ],
    )
    def k(ts_hbm, tr_hbm, snd_hbm, rcv_hbm, gs_hbm, gr_hbm,
          idx_s, idx_r, buf_s, buf_r, sem_s, sem_r):
        wid = lax.axis_index("s") * NC + lax.axis_index("c")
        w_base = wid * per_w

        @pl.loop(0, chunks)
        def _chunk(i):
            base = w_base + i * chunk
            pltpu.sync_copy(snd_hbm.at[pl.ds(base, chunk)], idx_s)
            pltpu.sync_copy(rcv_hbm.at[pl.ds(base, chunk)], idx_r)
            cs = pltpu.async_copy(ts_hbm.at[idx_s], buf_s, sem_s)
            cr = pltpu.async_copy(tr_hbm.at[idx_r], buf_r, sem_r)
            cs.wait()
            cr.wait()
            pltpu.sync_copy(buf_s, gs_hbm.at[pl.ds(base, chunk)])
            pltpu.sync_copy(buf_r, gr_hbm.at[pl.ds(base, chunk)])

    return k(ts, tr, snd, rcv)


def _edge_body(gs_ref, gr_ref, ed_ref, rcv_ref, wqe_ref, w1e_ref, b1_ref,
               gam_ref, bet_ref, w2_ref, b2_ref, outa_ref, outb_ref):
    gs = gs_ref[...]
    eq = jnp.dot(ed_ref[...], wqe_ref[...], preferred_element_type=_f32)
    eh = jnp.dot(ed_ref[...], w1e_ref[...], preferred_element_type=_f32)
    q = _unpack_hi(gs) + eq
    hpre = _unpack_lo(gs) + gr_ref[...] + eh + b1_ref[...]
    h = jnp.maximum(hpre, 0.0)
    mu = jnp.mean(h, axis=1, keepdims=True)
    var = jnp.mean((h - mu) * (h - mu), axis=1, keepdims=True)
    hln = (h - mu) / jnp.sqrt(var + 1e-5) * gam_ref[...] + bet_ref[...]
    lg = jnp.sum(hln * w2_ref[...], axis=1, keepdims=True) + b2_ref[...]
    w = jnp.exp(jnp.maximum(lg, 0.0))
    lane = lax.broadcasted_iota(jnp.int32, (1, 128), 1)
    oh = (lane == (rcv_ref[...] % 128)).astype(_f32)
    outa_ref[...] = q * w
    outb_ref[...] = w * oh


def _edge_compute(gs, gr, edges, rcv2d, wqe_t, w1e_t, b1, gamma, beta, w2, b2,
                  e_blk):
    e, de = edges.shape
    smalls = [wqe_t, w1e_t, b1.reshape(1, 128), gamma.reshape(1, 128),
              beta.reshape(1, 128), w2.reshape(1, 128), b2.reshape(1, 1)]
    small_specs = [
        pl.BlockSpec((de, 128), lambda i: (0, 0)),
        pl.BlockSpec((de, 128), lambda i: (0, 0)),
        pl.BlockSpec((1, 128), lambda i: (0, 0)),
        pl.BlockSpec((1, 128), lambda i: (0, 0)),
        pl.BlockSpec((1, 128), lambda i: (0, 0)),
        pl.BlockSpec((1, 128), lambda i: (0, 0)),
        pl.BlockSpec((1, 1), lambda i: (0, 0)),
    ]
    return pl.pallas_call(
        _edge_body,
        grid=(e // e_blk,),
        in_specs=[
            pl.BlockSpec((e_blk, 128), lambda i: (i, 0)),
            pl.BlockSpec((e_blk, 128), lambda i: (i, 0)),
            pl.BlockSpec((e_blk, de), lambda i: (i, 0)),
            pl.BlockSpec((e_blk, 1), lambda i: (i, 0)),
        ] + small_specs,
        out_specs=[pl.BlockSpec((e_blk, 128), lambda i: (i, 0)),
                   pl.BlockSpec((e_blk, 128), lambda i: (i, 0))],
        out_shape=[jax.ShapeDtypeStruct((e, 128), _f32),
                   jax.ShapeDtypeStruct((e, 128), _f32)],
    )(gs, gr, edges, rcv2d, *smalls)


def _sc_scatter(msga, msgb, rcv, chunk):
    e = rcv.shape[0]
    per_w = e // NW
    chunks = per_w // chunk
    zr = 128
    per_tile_rows = NP // NS
    zcopies = per_tile_rows // zr
    n_acc = NP + WR
    mesh = plsc.VectorSubcoreMesh(
        core_axis_name="c", subcore_axis_name="s", num_cores=NC, num_subcores=NS)

    @functools.partial(
        pl.kernel,
        out_type=jax.ShapeDtypeStruct((NC, n_acc, 128), _f32),
        mesh=mesh,
        scratch_types=[
            pltpu.VMEM((chunk,), jnp.int32),
            pltpu.VMEM((chunk,), jnp.int32),
            pltpu.VMEM((chunk, 128), _f32),
            pltpu.VMEM((chunk, 128), _f32),
            pltpu.VMEM((zr, 128), _f32),
            pltpu.VMEM_SHARED((n_acc, 128), _f32),
        ],
    )
    def k(msga_hbm, msgb_hbm, rcv_hbm, out_hbm, idx, idx2, bufa, bufb, zbuf, acc):
        cid = lax.axis_index("c")
        sid = lax.axis_index("s")
        wid = sid * NC + cid
        w_base = wid * per_w
        zeros16 = jnp.zeros((16,), _f32)

        @pl.loop(0, zr)
        def _zrow(r):
            for kk in range(128 // 16):
                zbuf[r, pl.ds(kk * 16, 16)] = zeros16

        @pl.loop(0, zcopies)
        def _zcp(zi):
            pltpu.sync_copy(zbuf, acc.at[pl.ds(sid * per_tile_rows + zi * zr, zr)])

        @pl.when(sid == 0)
        def _zw():
            pltpu.sync_copy(zbuf.at[pl.ds(0, WR)], acc.at[pl.ds(NP, WR)])

        plsc.subcore_barrier()

        @pl.loop(0, chunks)
        def _chunk(i):
            base = w_base + i * chunk
            pltpu.sync_copy(rcv_hbm.at[pl.ds(base, chunk)], idx)
            pltpu.sync_copy(msga_hbm.at[pl.ds(base, chunk)], bufa)
            pltpu.sync_copy(msgb_hbm.at[pl.ds(base, chunk)], bufb)
            for kk in range(chunk // 16):
                v = idx[pl.ds(kk * 16, 16)]
                idx2[pl.ds(kk * 16, 16)] = NP + lax.shift_right_logical(v, 7)
            pltpu.sync_copy(bufa, acc.at[idx], add=True)
            pltpu.sync_copy(bufb, acc.at[idx2], add=True)

        plsc.subcore_barrier()

        @pl.loop(0, zcopies)
        def _dump(zi):
            row = sid * per_tile_rows + zi * zr
            pltpu.sync_copy(acc.at[pl.ds(row, zr)], out_hbm.at[cid, pl.ds(row, zr)])

        @pl.when(sid == 0)
        def _dw():
            pltpu.sync_copy(acc.at[pl.ds(NP, WR)], out_hbm.at[cid, pl.ds(NP, WR)])

    return k(msga, msgb, rcv)


def _final_body(parts_ref, wcol_ref, qr_ref, bq_ref, out_ref):
    num = parts_ref[0] + parts_ref[1]
    ws = wcol_ref[...]
    val = (num + ws * (qr_ref[...] + bq_ref[...])) / (ws + 1e-10)
    out_ref[...] = jnp.where(val >= 0.0, val, 0.01 * val)


def _finalize(parts, wcol, qr, bq, n_blk):
    n = qr.shape[0]
    return pl.pallas_call(
        _final_body,
        grid=(n // n_blk,),
        in_specs=[
            pl.BlockSpec((2, n_blk, 128), lambda i: (0, i, 0)),
            pl.BlockSpec((n_blk, 1), lambda i: (i, 0)),
            pl.BlockSpec((n_blk, 128), lambda i: (i, 0)),
            pl.BlockSpec((1, 128), lambda i: (0, 0)),
        ],
        out_specs=pl.BlockSpec((n_blk, 128), lambda i: (i, 0)),
        out_shape=jax.ShapeDtypeStruct((n, 128), _f32),
    )(parts, wcol, qr, bq.reshape(1, 128))


def kernel(nodes, edge_index, edges, Wq, bq, W1, b1, gamma, beta, W2, b2):
    n, d = nodes.shape
    e = edge_index.shape[1]
    wall = jnp.concatenate(
        [Wq[:, :d].T, W1[:, :d].T, W1[:, d:2 * d].T, Wq[:, d:2 * d].T], axis=1)
    wqe_t = Wq[:, 2 * d:].T
    w1e_t = W1[:, 2 * d:].T
    snd = edge_index[0]
    rcv = edge_index[1]
    nodes_p = jnp.pad(nodes, ((0, NP - n), (0, 0)))

    ts, tr, qr = _node_tables(nodes_p, wall, 2048)
    gs, gr = _sc_gather(ts, tr, snd, rcv, 80)
    msga, msgb = _edge_compute(gs, gr, edges, rcv.reshape(e, 1), wqe_t, w1e_t,
                               b1, gamma, beta, W2[0], b2, 2560)
    parts = _sc_scatter(msga, msgb, rcv, 80)
    wcol = (parts[0, NP:] + parts[1, NP:]).reshape(NP)[:, None]
    out = _finalize(parts, wcol, qr, bq, 2048)
    return out[:n]
